# Initial kernel scaffold; baseline (speedup 1.0000x reference)
import jax
import jax.numpy as jnp
from jax import lax
from jax.experimental import pallas as pl

V_MIN = -10.0
V_MAX = 10.0
NUM_ATOMS = 51
DELTA_Z = (V_MAX - V_MIN) / (NUM_ATOMS - 1)

_ROWS = 16384
_COLS = 64
_BLOCK_R = 256


def _c51_block_kernel(x_ref, out_ref):
    x = x_ref[...]
    t = jnp.clip(x, V_MIN, V_MAX)
    b = (t - V_MIN) / DELTA_Z
    l = jnp.floor(b)
    b3 = b[:, :, None]
    l3 = l[:, :, None]
    k = lax.broadcasted_iota(jnp.float32, (x.shape[0], x.shape[1], NUM_ATOMS), 2)
    wl = (l3 + 1.0) - b3
    wu = b3 - l3
    out = jnp.where(k == l3, wl, 0.0) + jnp.where(k == l3 + 1.0, wu, 0.0)
    out_ref[...] = out


def kernel(scalar):
    grid = (_ROWS // _BLOCK_R,)
    return pl.pallas_call(
        _c51_block_kernel,
        grid=grid,
        in_specs=[pl.BlockSpec((_BLOCK_R, _COLS), lambda i: (i, 0))],
        out_specs=pl.BlockSpec((_BLOCK_R, _COLS, NUM_ATOMS), lambda i: (i, 0, 0)),
        out_shape=jax.ShapeDtypeStruct((_ROWS, _COLS, NUM_ATOMS), scalar.dtype),
    )(scalar)


# speedup vs baseline: 11.8837x; 11.8837x over previous
import jax
import jax.numpy as jnp
from jax import lax
from jax.experimental import pallas as pl

V_MIN = -10.0
V_MAX = 10.0
NUM_ATOMS = 51
DELTA_Z = (V_MAX - V_MIN) / (NUM_ATOMS - 1)

_ROWS = 16384
_COLS = 64
_BLOCK_R = 256


def _c51_block_kernel(x_ref, out_ref):
    x = x_ref[...]
    t = jnp.clip(x, V_MIN, V_MAX)
    b = (t - V_MIN) / DELTA_Z
    l = jnp.floor(b)
    li = l.astype(jnp.int32)
    b3 = b[:, :, None]
    l3 = l[:, :, None]
    li3 = li[:, :, None]
    k = lax.broadcasted_iota(jnp.int32, (x.shape[0], x.shape[1], NUM_ATOMS), 2)
    wl = (l3 + 1.0) - b3
    wu = b3 - l3
    out = jnp.where(k == li3, wl, 0.0) + jnp.where(k == li3 + 1, wu, 0.0)
    out_ref[...] = out


def kernel(scalar):
    grid = (_ROWS // _BLOCK_R,)
    return pl.pallas_call(
        _c51_block_kernel,
        grid=grid,
        in_specs=[pl.BlockSpec((_BLOCK_R, _COLS), lambda i: (i, 0))],
        out_specs=pl.BlockSpec((_BLOCK_R, _COLS, NUM_ATOMS), lambda i: (i, 0, 0)),
        out_shape=jax.ShapeDtypeStruct((_ROWS, _COLS, NUM_ATOMS), scalar.dtype),
    )(scalar)


# speedup vs baseline: 14.8773x; 1.2519x over previous
import jax
import jax.numpy as jnp
from jax import lax
from jax.experimental import pallas as pl

V_MIN = -10.0
V_MAX = 10.0
NUM_ATOMS = 51
DELTA_Z = (V_MAX - V_MIN) / (NUM_ATOMS - 1)

_ROWS = 16384
_COLS = 64
_OUTW = _COLS * NUM_ATOMS
_BLOCK_R = 256


def _c51_block_kernel(g_ref, x_ref, out_ref):
    x = x_ref[...]
    t = jnp.clip(x, V_MIN, V_MAX)
    b = (t - V_MIN) / DELTA_Z
    b2 = jnp.dot(b, g_ref[...], preferred_element_type=jnp.float32)
    c = lax.broadcasted_iota(jnp.int32, (x.shape[0], _OUTW), 1)
    k2 = c - NUM_ATOMS * (c // NUM_ATOMS)
    l2 = jnp.floor(b2)
    li = l2.astype(jnp.int32)
    wl = (l2 + 1.0) - b2
    wu = b2 - l2
    out_ref[...] = jnp.where(k2 == li, wl, 0.0) + jnp.where(k2 == li + 1, wu, 0.0)


def kernel(scalar):
    j = jnp.arange(_OUTW, dtype=jnp.int32) // NUM_ATOMS
    g = (j[None, :] == jnp.arange(_COLS, dtype=jnp.int32)[:, None]).astype(jnp.float32)
    out2d = pl.pallas_call(
        _c51_block_kernel,
        grid=(_ROWS // _BLOCK_R,),
        in_specs=[
            pl.BlockSpec((_COLS, _OUTW), lambda i: (0, 0)),
            pl.BlockSpec((_BLOCK_R, _COLS), lambda i: (i, 0)),
        ],
        out_specs=pl.BlockSpec((_BLOCK_R, _OUTW), lambda i: (i, 0)),
        out_shape=jax.ShapeDtypeStruct((_ROWS, _OUTW), scalar.dtype),
    )(g, scalar)
    return out2d.reshape(_ROWS, _COLS, NUM_ATOMS)
